# baseline (speedup 1.0000x reference)
import functools

import jax
import jax.numpy as jnp
from jax import lax
from jax.experimental import pallas as pl
from jax.experimental.pallas import tpu as pltpu
from jax.experimental.pallas import tpu_sc as plsc

N_NODES = 10000
NPAD = 10240
N_EDGES = 320000
D = 128

NC = 2
NS = 16
NW = NC * NS
EPW = N_EDGES // NW
K = 128
CPT = 80
E_PAD = NW * CPT * K
RPT = NPAD // NS


def _agg_body(h_hbm, src_hbm, dst_hbm, zero_hbm, out_hbm,
              sidx0, sidx1, didx0, didx1, rows0, rows1, acc_sh,
              gs0, gs1, is0, is1, id0, id1):
    c = lax.axis_index("c")
    s = lax.axis_index("s")
    wid = s * NC + c
    row0 = s * RPT

    @pl.when(c == 0)
    def _():
        pltpu.sync_copy(h_hbm.at[pl.ds(row0, RPT)], acc_sh.at[pl.ds(row0, RPT)])

    @pl.when(c != 0)
    def _():
        pltpu.sync_copy(zero_hbm.at[pl.ds(row0, RPT)], acc_sh.at[pl.ds(row0, RPT)])

    sidx = (sidx0, sidx1)
    didx = (didx0, didx1)
    rows = (rows0, rows1)
    gsem = (gs0, gs1)
    isem = (is0, is1)
    dsem = (id0, id1)
    my_src = src_hbm.at[wid]
    my_dst = dst_hbm.at[wid]

    pltpu.sync_copy(my_src.at[0], sidx[0])
    pltpu.sync_copy(my_dst.at[0], didx[0])
    pltpu.async_copy(h_hbm.at[sidx[0]], rows[0], gsem[0])
    pltpu.async_copy(my_src.at[1], sidx[1], isem[1])
    pltpu.async_copy(my_dst.at[1], didx[1], dsem[1])
    plsc.subcore_barrier()

    def group(g, carry):
        for b in range(2):
            i = g * 2 + b
            o = 1 - b

            pltpu.make_async_copy(h_hbm.at[sidx[b]], rows[b], gsem[b]).wait()

            @pl.when(i + 1 < CPT)
            def _():
                pltpu.make_async_copy(my_src.at[i + 1], sidx[o],
                                      isem[o]).wait()
                pltpu.make_async_copy(my_dst.at[i + 1], didx[o],
                                      dsem[o]).wait()
                pltpu.async_copy(h_hbm.at[sidx[o]], rows[o], gsem[o])

            pltpu.sync_copy(rows[b], acc_sh.at[didx[b]], add=True)

            @pl.when(i + 2 < CPT)
            def _():
                pltpu.async_copy(my_src.at[i + 2], sidx[b], isem[b])
                pltpu.async_copy(my_dst.at[i + 2], didx[b], dsem[b])
        return carry

    lax.fori_loop(0, CPT // 2, group, 0)
    plsc.subcore_barrier()
    pltpu.sync_copy(acc_sh.at[pl.ds(row0, RPT)],
                    out_hbm.at[c].at[pl.ds(row0, RPT)])


_gin_agg = functools.partial(
    pl.kernel,
    out_type=jax.ShapeDtypeStruct((NC, NPAD, D), jnp.float32),
    mesh=plsc.VectorSubcoreMesh(core_axis_name="c", subcore_axis_name="s",
                                num_cores=NC, num_subcores=NS),
    scratch_types=[
        pltpu.VMEM((K,), jnp.int32),
        pltpu.VMEM((K,), jnp.int32),
        pltpu.VMEM((K,), jnp.int32),
        pltpu.VMEM((K,), jnp.int32),
        pltpu.VMEM((K, D), jnp.float32),
        pltpu.VMEM((K, D), jnp.float32),
        pltpu.VMEM_SHARED((NPAD, D), jnp.float32),
        pltpu.SemaphoreType.DMA,
        pltpu.SemaphoreType.DMA,
        pltpu.SemaphoreType.DMA,
        pltpu.SemaphoreType.DMA,
        pltpu.SemaphoreType.DMA,
        pltpu.SemaphoreType.DMA,
    ],
)(_agg_body)


R = 1000


def _elu(x):
    return jnp.where(x > 0, x, jnp.exp(jnp.minimum(x, 0.0)) - 1.0)


def _mlp_mid_body(p_ref, W1_ref, b1_ref, W2_ref, b2_ref, out_ref):
    z = p_ref[0] + p_ref[1]
    z = jnp.maximum(
        jnp.dot(z, W1_ref[...], preferred_element_type=jnp.float32)
        + b1_ref[...], 0.0)
    h = jnp.dot(z, W2_ref[...], preferred_element_type=jnp.float32) + b2_ref[...]
    out_ref[...] = _elu(h)


def _mlp_final_body(p_ref, W1_ref, b1_ref, W2_ref, b2_ref,
                    lw_ref, lb_ref, out_ref):
    z = p_ref[0] + p_ref[1]
    z = jnp.maximum(
        jnp.dot(z, W1_ref[...], preferred_element_type=jnp.float32)
        + b1_ref[...], 0.0)
    h = jnp.dot(z, W2_ref[...], preferred_element_type=jnp.float32) + b2_ref[...]
    h = _elu(h)
    o = jnp.dot(h, lw_ref[...], preferred_element_type=jnp.float32) + lb_ref[...]
    out_ref[...] = 1.0 / (1.0 + jnp.exp(-o))


_P_SPEC = pl.BlockSpec((NC, R, D), lambda i: (0, i, 0))
_W_SPEC = pl.BlockSpec((D, D), lambda i: (0, 0))
_B_SPEC = pl.BlockSpec((1, D), lambda i: (0, 0))

_mlp_mid = pl.pallas_call(
    _mlp_mid_body,
    grid=(N_NODES // R,),
    in_specs=[_P_SPEC, _W_SPEC, _B_SPEC, _W_SPEC, _B_SPEC],
    out_specs=pl.BlockSpec((R, D), lambda i: (i, 0)),
    out_shape=jax.ShapeDtypeStruct((NPAD, D), jnp.float32),
)

_mlp_final = pl.pallas_call(
    _mlp_final_body,
    grid=(N_NODES // R,),
    in_specs=[_P_SPEC, _W_SPEC, _B_SPEC, _W_SPEC, _B_SPEC,
              pl.BlockSpec((D, 1), lambda i: (0, 0)),
              pl.BlockSpec((1, 1), lambda i: (0, 0))],
    out_specs=pl.BlockSpec((R, 1), lambda i: (i, 0)),
    out_shape=jax.ShapeDtypeStruct((N_NODES, 1), jnp.float32),
)


def kernel(x, edge_index, conv0_W1, conv0_b1, conv0_W2, conv0_b2,
           conv1_W1, conv1_b1, conv1_W2, conv1_b2, lin_W, lin_b):
    pad = E_PAD - N_EDGES
    src = jnp.concatenate(
        [edge_index[0].astype(jnp.int32), jnp.zeros((pad,), jnp.int32)]
    ).reshape(NW, CPT, K)
    dst = jnp.concatenate(
        [edge_index[1].astype(jnp.int32),
         jnp.full((pad,), N_NODES, jnp.int32)]
    ).reshape(NW, CPT, K)
    zeros = jnp.zeros((NPAD, D), jnp.float32)
    x_pad = jnp.concatenate(
        [x, jnp.zeros((NPAD - N_NODES, D), jnp.float32)], axis=0)

    p = _gin_agg(x_pad, src, dst, zeros)
    h1 = _mlp_mid(p, conv0_W1, conv0_b1.reshape(1, D),
                  conv0_W2, conv0_b2.reshape(1, D))
    p = _gin_agg(h1, src, dst, zeros)
    out = _mlp_final(p, conv1_W1, conv1_b1.reshape(1, D),
                     conv1_W2, conv1_b2.reshape(1, D),
                     lin_W, lin_b.reshape(1, 1))
    return out.reshape(N_NODES)


# speedup vs baseline: 2.8839x; 2.8839x over previous
import functools

import jax
import jax.numpy as jnp
from jax import lax
from jax.experimental import pallas as pl
from jax.experimental.pallas import tpu as pltpu
from jax.experimental.pallas import tpu_sc as plsc

N_NODES = 10000
NPAD = 10240
N_EDGES = 320000
D = 128

NC = 2
NS = 16
NW = NC * NS
EPW = N_EDGES // NW
K = 128
CPT = 80
E_PAD = NW * CPT * K
RPT = NPAD // NS


def _agg_body(h_hbm, src_hbm, dst_hbm, zero_hbm, out_hbm,
              sidx0, sidx1, didx0, didx1, rows0, rows1, acc_sh,
              gs0, gs1, is0, is1, id0, id1):
    c = lax.axis_index("c")
    s = lax.axis_index("s")
    wid = s * NC + c
    row0 = s * RPT

    @pl.when(c == 0)
    def _():
        pltpu.sync_copy(h_hbm.at[pl.ds(row0, RPT)], acc_sh.at[pl.ds(row0, RPT)])

    @pl.when(c != 0)
    def _():
        pltpu.sync_copy(zero_hbm.at[pl.ds(row0, RPT)], acc_sh.at[pl.ds(row0, RPT)])

    sidx = (sidx0, sidx1)
    didx = (didx0, didx1)
    rows = (rows0, rows1)
    gsem = (gs0, gs1)
    isem = (is0, is1)
    dsem = (id0, id1)
    my_src = src_hbm.at[wid]
    my_dst = dst_hbm.at[wid]

    pltpu.sync_copy(my_src.at[0], sidx[0])
    pltpu.sync_copy(my_dst.at[0], didx[0])
    pltpu.async_copy(h_hbm.at[sidx[0]], rows[0], gsem[0])
    pltpu.async_copy(my_src.at[1], sidx[1], isem[1])
    pltpu.async_copy(my_dst.at[1], didx[1], dsem[1])
    plsc.subcore_barrier()

    def group(g, carry):
        for b in range(2):
            i = g * 2 + b
            o = 1 - b

            pltpu.make_async_copy(h_hbm.at[sidx[b]], rows[b], gsem[b]).wait()

            @pl.when(i + 1 < CPT)
            def _():
                pltpu.make_async_copy(my_src.at[i + 1], sidx[o],
                                      isem[o]).wait()
                pltpu.make_async_copy(my_dst.at[i + 1], didx[o],
                                      dsem[o]).wait()
                pltpu.async_copy(h_hbm.at[sidx[o]], rows[o], gsem[o])

            pltpu.sync_copy(rows[b], acc_sh.at[didx[b]], add=True)

            @pl.when(i + 2 < CPT)
            def _():
                pltpu.async_copy(my_src.at[i + 2], sidx[b], isem[b])
                pltpu.async_copy(my_dst.at[i + 2], didx[b], dsem[b])
        return carry

    lax.fori_loop(0, CPT // 2, group, 0)
    plsc.subcore_barrier()
    pltpu.sync_copy(acc_sh.at[pl.ds(row0, RPT)],
                    out_hbm.at[c].at[pl.ds(row0, RPT)])


_gin_agg = functools.partial(
    pl.kernel,
    out_type=jax.ShapeDtypeStruct((NC, NPAD, D), jnp.float32),
    mesh=plsc.VectorSubcoreMesh(core_axis_name="c", subcore_axis_name="s",
                                num_cores=NC, num_subcores=NS),
    scratch_types=[
        pltpu.VMEM((K,), jnp.int32),
        pltpu.VMEM((K,), jnp.int32),
        pltpu.VMEM((K,), jnp.int32),
        pltpu.VMEM((K,), jnp.int32),
        pltpu.VMEM((K, D), jnp.float32),
        pltpu.VMEM((K, D), jnp.float32),
        pltpu.VMEM_SHARED((NPAD, D), jnp.float32),
        pltpu.SemaphoreType.DMA,
        pltpu.SemaphoreType.DMA,
        pltpu.SemaphoreType.DMA,
        pltpu.SemaphoreType.DMA,
        pltpu.SemaphoreType.DMA,
        pltpu.SemaphoreType.DMA,
    ],
)(_agg_body)


R = 1000


def _elu(x):
    return jnp.where(x > 0, x, jnp.exp(jnp.minimum(x, 0.0)) - 1.0)


def _mlp_mid_body(p_ref, W1_ref, b1_ref, W2_ref, b2_ref, out_ref):
    z = p_ref[0] + p_ref[1]
    z = jnp.maximum(
        jnp.dot(z, W1_ref[...], preferred_element_type=jnp.float32)
        + b1_ref[...], 0.0)
    h = jnp.dot(z, W2_ref[...], preferred_element_type=jnp.float32) + b2_ref[...]
    out_ref[...] = _elu(h)


def _mlp_final_body(p_ref, W1_ref, b1_ref, W2_ref, b2_ref,
                    lw_ref, lb_ref, out_ref):
    z = p_ref[0] + p_ref[1]
    z = jnp.maximum(
        jnp.dot(z, W1_ref[...], preferred_element_type=jnp.float32)
        + b1_ref[...], 0.0)
    h = jnp.dot(z, W2_ref[...], preferred_element_type=jnp.float32) + b2_ref[...]
    h = _elu(h)
    o = jnp.dot(h, lw_ref[...], preferred_element_type=jnp.float32) + lb_ref[...]
    out_ref[...] = 1.0 / (1.0 + jnp.exp(-o))


_P_SPEC = pl.BlockSpec((NC, R, D), lambda i: (0, i, 0))
_W_SPEC = pl.BlockSpec((D, D), lambda i: (0, 0))
_B_SPEC = pl.BlockSpec((1, D), lambda i: (0, 0))

_mlp_mid = pl.pallas_call(
    _mlp_mid_body,
    grid=(N_NODES // R,),
    in_specs=[_P_SPEC, _W_SPEC, _B_SPEC, _W_SPEC, _B_SPEC],
    out_specs=pl.BlockSpec((R, D), lambda i: (i, 0)),
    out_shape=jax.ShapeDtypeStruct((NPAD, D), jnp.float32),
)

_mlp_final = pl.pallas_call(
    _mlp_final_body,
    grid=(N_NODES // R,),
    in_specs=[_P_SPEC, _W_SPEC, _B_SPEC, _W_SPEC, _B_SPEC,
              pl.BlockSpec((D, 1), lambda i: (0, 0)),
              pl.BlockSpec((1, 1), lambda i: (0, 0))],
    out_specs=pl.BlockSpec((R, 1), lambda i: (i, 0)),
    out_shape=jax.ShapeDtypeStruct((N_NODES, 1), jnp.float32),
)


def kernel(x, edge_index, conv0_W1, conv0_b1, conv0_W2, conv0_b2,
           conv1_W1, conv1_b1, conv1_W2, conv1_b2, lin_W, lin_b):
    pad = E_PAD - N_EDGES
    pad_idx = jax.lax.iota(jnp.int32, pad)
    src = jnp.concatenate(
        [edge_index[0].astype(jnp.int32), pad_idx % N_NODES]
    ).reshape(NW, CPT, K)
    dst = jnp.concatenate(
        [edge_index[1].astype(jnp.int32),
         N_NODES + pad_idx % (NPAD - N_NODES)]
    ).reshape(NW, CPT, K)
    zeros = jnp.zeros((NPAD, D), jnp.float32)
    x_pad = jnp.concatenate(
        [x, jnp.zeros((NPAD - N_NODES, D), jnp.float32)], axis=0)

    p = _gin_agg(x_pad, src, dst, zeros)
    h1 = _mlp_mid(p, conv0_W1, conv0_b1.reshape(1, D),
                  conv0_W2, conv0_b2.reshape(1, D))
    p = _gin_agg(h1, src, dst, zeros)
    out = _mlp_final(p, conv1_W1, conv1_b1.reshape(1, D),
                     conv1_W2, conv1_b2.reshape(1, D),
                     lin_W, lin_b.reshape(1, 1))
    return out.reshape(N_NODES)


# speedup vs baseline: 2.8923x; 1.0029x over previous
import functools

import jax
import jax.numpy as jnp
from jax import lax
from jax.experimental import pallas as pl
from jax.experimental.pallas import tpu as pltpu
from jax.experimental.pallas import tpu_sc as plsc

N_NODES = 10000
NPAD = 10240
N_EDGES = 320000
D = 128

NC = 2
NS = 16
NW = NC * NS
EPW = N_EDGES // NW
K = 128
CPT = 80
E_PAD = NW * CPT * K
RPT = NPAD // NS


def _agg_body(h_hbm, src_hbm, dst_hbm, zero_hbm, out_hbm,
              sidx0, sidx1, didx0, didx1, didx2, didx3, rows0, rows1, acc_sh,
              gs0, gs1, is0, is1, id0, id1, id2, id3, ss0, ss1):
    c = lax.axis_index("c")
    s = lax.axis_index("s")
    wid = s * NC + c
    row0 = s * RPT

    @pl.when(c == 0)
    def _():
        pltpu.sync_copy(h_hbm.at[pl.ds(row0, RPT)], acc_sh.at[pl.ds(row0, RPT)])

    @pl.when(c != 0)
    def _():
        pltpu.sync_copy(zero_hbm.at[pl.ds(row0, RPT)], acc_sh.at[pl.ds(row0, RPT)])

    sidx = (sidx0, sidx1)
    didx = (didx0, didx1, didx2, didx3)
    rows = (rows0, rows1)
    gsem = (gs0, gs1)
    isem = (is0, is1)
    dsem = (id0, id1, id2, id3)
    ssem = (ss0, ss1)
    my_src = src_hbm.at[wid]
    my_dst = dst_hbm.at[wid]

    pltpu.sync_copy(my_src.at[0], sidx[0])
    pltpu.sync_copy(my_dst.at[0], didx[0])
    pltpu.async_copy(h_hbm.at[sidx[0]], rows[0], gsem[0])
    pltpu.async_copy(my_src.at[1], sidx[1], isem[1])
    pltpu.async_copy(my_dst.at[1], didx[1], dsem[1])
    plsc.subcore_barrier()

    def group(g, carry):
        for b4 in range(4):
            i = g * 4 + b4
            b = b4 % 2
            o = 1 - b

            pltpu.make_async_copy(h_hbm.at[sidx[b]], rows[b], gsem[b]).wait()

            @pl.when(i >= 1)
            def _():
                pltpu.make_async_copy(
                    rows[o], acc_sh.at[didx[(b4 - 1) % 4]], ssem[o]).wait()

            @pl.when(i + 1 < CPT)
            def _():
                pltpu.make_async_copy(my_src.at[i + 1], sidx[o],
                                      isem[o]).wait()
                pltpu.make_async_copy(my_dst.at[i + 1], didx[(b4 + 1) % 4],
                                      dsem[(b4 + 1) % 4]).wait()
                pltpu.async_copy(h_hbm.at[sidx[o]], rows[o], gsem[o])

            pltpu.async_copy(rows[b], acc_sh.at[didx[b4]], ssem[b], add=True)

            @pl.when(i + 2 < CPT)
            def _():
                pltpu.async_copy(my_src.at[i + 2], sidx[b], isem[b])
                pltpu.async_copy(my_dst.at[i + 2], didx[(b4 + 2) % 4],
                                 dsem[(b4 + 2) % 4])
        return carry

    lax.fori_loop(0, CPT // 4, group, 0)
    pltpu.make_async_copy(
        rows[(CPT - 1) % 2], acc_sh.at[didx[(CPT - 1) % 4]],
        ssem[(CPT - 1) % 2]).wait()
    plsc.subcore_barrier()
    pltpu.sync_copy(acc_sh.at[pl.ds(row0, RPT)],
                    out_hbm.at[c].at[pl.ds(row0, RPT)])


_gin_agg = functools.partial(
    pl.kernel,
    out_type=jax.ShapeDtypeStruct((NC, NPAD, D), jnp.float32),
    mesh=plsc.VectorSubcoreMesh(core_axis_name="c", subcore_axis_name="s",
                                num_cores=NC, num_subcores=NS),
    scratch_types=[
        pltpu.VMEM((K,), jnp.int32),
        pltpu.VMEM((K,), jnp.int32),
        pltpu.VMEM((K,), jnp.int32),
        pltpu.VMEM((K,), jnp.int32),
        pltpu.VMEM((K,), jnp.int32),
        pltpu.VMEM((K,), jnp.int32),
        pltpu.VMEM((K, D), jnp.float32),
        pltpu.VMEM((K, D), jnp.float32),
        pltpu.VMEM_SHARED((NPAD, D), jnp.float32),
        pltpu.SemaphoreType.DMA,
        pltpu.SemaphoreType.DMA,
        pltpu.SemaphoreType.DMA,
        pltpu.SemaphoreType.DMA,
        pltpu.SemaphoreType.DMA,
        pltpu.SemaphoreType.DMA,
        pltpu.SemaphoreType.DMA,
        pltpu.SemaphoreType.DMA,
        pltpu.SemaphoreType.DMA,
        pltpu.SemaphoreType.DMA,
    ],
)(_agg_body)


R = 1000


def _elu(x):
    return jnp.where(x > 0, x, jnp.exp(jnp.minimum(x, 0.0)) - 1.0)


def _mlp_mid_body(p_ref, W1_ref, b1_ref, W2_ref, b2_ref, out_ref):
    z = p_ref[0] + p_ref[1]
    z = jnp.maximum(
        jnp.dot(z, W1_ref[...], preferred_element_type=jnp.float32)
        + b1_ref[...], 0.0)
    h = jnp.dot(z, W2_ref[...], preferred_element_type=jnp.float32) + b2_ref[...]
    out_ref[...] = _elu(h)


def _mlp_final_body(p_ref, W1_ref, b1_ref, W2_ref, b2_ref,
                    lw_ref, lb_ref, out_ref):
    z = p_ref[0] + p_ref[1]
    z = jnp.maximum(
        jnp.dot(z, W1_ref[...], preferred_element_type=jnp.float32)
        + b1_ref[...], 0.0)
    h = jnp.dot(z, W2_ref[...], preferred_element_type=jnp.float32) + b2_ref[...]
    h = _elu(h)
    o = jnp.dot(h, lw_ref[...], preferred_element_type=jnp.float32) + lb_ref[...]
    out_ref[...] = 1.0 / (1.0 + jnp.exp(-o))


_P_SPEC = pl.BlockSpec((NC, R, D), lambda i: (0, i, 0))
_W_SPEC = pl.BlockSpec((D, D), lambda i: (0, 0))
_B_SPEC = pl.BlockSpec((1, D), lambda i: (0, 0))

_mlp_mid = pl.pallas_call(
    _mlp_mid_body,
    grid=(N_NODES // R,),
    in_specs=[_P_SPEC, _W_SPEC, _B_SPEC, _W_SPEC, _B_SPEC],
    out_specs=pl.BlockSpec((R, D), lambda i: (i, 0)),
    out_shape=jax.ShapeDtypeStruct((NPAD, D), jnp.float32),
)

_mlp_final = pl.pallas_call(
    _mlp_final_body,
    grid=(N_NODES // R,),
    in_specs=[_P_SPEC, _W_SPEC, _B_SPEC, _W_SPEC, _B_SPEC,
              pl.BlockSpec((D, 1), lambda i: (0, 0)),
              pl.BlockSpec((1, 1), lambda i: (0, 0))],
    out_specs=pl.BlockSpec((R, 1), lambda i: (i, 0)),
    out_shape=jax.ShapeDtypeStruct((N_NODES, 1), jnp.float32),
)


def kernel(x, edge_index, conv0_W1, conv0_b1, conv0_W2, conv0_b2,
           conv1_W1, conv1_b1, conv1_W2, conv1_b2, lin_W, lin_b):
    pad = E_PAD - N_EDGES
    pad_idx = jax.lax.iota(jnp.int32, pad)
    src = jnp.concatenate(
        [edge_index[0].astype(jnp.int32), pad_idx % N_NODES]
    ).reshape(NW, CPT, K)
    dst = jnp.concatenate(
        [edge_index[1].astype(jnp.int32),
         N_NODES + pad_idx % (NPAD - N_NODES)]
    ).reshape(NW, CPT, K)
    zeros = jnp.zeros((NPAD, D), jnp.float32)
    x_pad = jnp.concatenate(
        [x, jnp.zeros((NPAD - N_NODES, D), jnp.float32)], axis=0)

    p = _gin_agg(x_pad, src, dst, zeros)
    h1 = _mlp_mid(p, conv0_W1, conv0_b1.reshape(1, D),
                  conv0_W2, conv0_b2.reshape(1, D))
    p = _gin_agg(h1, src, dst, zeros)
    out = _mlp_final(p, conv1_W1, conv1_b1.reshape(1, D),
                     conv1_W2, conv1_b2.reshape(1, D),
                     lin_W, lin_b.reshape(1, 1))
    return out.reshape(N_NODES)


# speedup vs baseline: 2.9071x; 1.0051x over previous
import functools

import jax
import jax.numpy as jnp
from jax import lax
from jax.experimental import pallas as pl
from jax.experimental.pallas import tpu as pltpu
from jax.experimental.pallas import tpu_sc as plsc

N_NODES = 10000
NPAD = 10240
N_EDGES = 320000
D = 128

NC = 2
NS = 16
NW = NC * NS
EPW = N_EDGES // NW
K = 128
CPT = 80
E_PAD = NW * CPT * K
RPT = NPAD // NS
LAST_RPT = N_NODES - (NS - 1) * RPT


def _agg_body(h_hbm, e_hbm, zero_hbm, out_hbm,
              ei0, ei1, ei2, ei3, rows0, rows1, acc_sh,
              gs0, gs1, es0, es1, es2, es3, ss0, ss1):
    c = lax.axis_index("c")
    s = lax.axis_index("s")
    wid = s * NC + c
    row0 = s * RPT

    @pl.when(c == 0)
    def _():
        @pl.when(s < NS - 1)
        def _():
            pltpu.sync_copy(h_hbm.at[pl.ds(row0, RPT)],
                            acc_sh.at[pl.ds(row0, RPT)])

        @pl.when(s == NS - 1)
        def _():
            pltpu.sync_copy(h_hbm.at[pl.ds(row0, LAST_RPT)],
                            acc_sh.at[pl.ds(row0, LAST_RPT)])

    @pl.when(c != 0)
    def _():
        pltpu.sync_copy(zero_hbm.at[pl.ds(row0, RPT)],
                        acc_sh.at[pl.ds(row0, RPT)])

    eidx = (ei0, ei1, ei2, ei3)
    esem = (es0, es1, es2, es3)
    rows = (rows0, rows1)
    gsem = (gs0, gs1)
    ssem = (ss0, ss1)
    my_e = e_hbm.at[wid]

    pltpu.sync_copy(my_e.at[0], eidx[0])
    pltpu.async_copy(h_hbm.at[eidx[0].at[0]], rows[0], gsem[0])
    pltpu.async_copy(my_e.at[1], eidx[1], esem[1])
    plsc.subcore_barrier()

    def group(g, carry):
        for b4 in range(4):
            i = g * 4 + b4
            b = b4 % 2
            o = 1 - b

            pltpu.make_async_copy(
                h_hbm.at[eidx[b4].at[0]], rows[b], gsem[b]).wait()

            @pl.when(i >= 1)
            def _():
                pltpu.make_async_copy(
                    rows[o], acc_sh.at[eidx[(b4 - 1) % 4].at[1]],
                    ssem[o]).wait()

            @pl.when(i + 1 < CPT)
            def _():
                pltpu.make_async_copy(my_e.at[i + 1], eidx[(b4 + 1) % 4],
                                      esem[(b4 + 1) % 4]).wait()
                pltpu.async_copy(h_hbm.at[eidx[(b4 + 1) % 4].at[0]],
                                 rows[o], gsem[o])

            pltpu.async_copy(rows[b], acc_sh.at[eidx[b4].at[1]],
                             ssem[b], add=True)

            @pl.when(i + 2 < CPT)
            def _():
                pltpu.async_copy(my_e.at[i + 2], eidx[(b4 + 2) % 4],
                                 esem[(b4 + 2) % 4])
        return carry

    lax.fori_loop(0, CPT // 4, group, 0)
    pltpu.make_async_copy(
        rows[(CPT - 1) % 2], acc_sh.at[eidx[(CPT - 1) % 4].at[1]],
        ssem[(CPT - 1) % 2]).wait()
    plsc.subcore_barrier()
    pltpu.sync_copy(acc_sh.at[pl.ds(row0, RPT)],
                    out_hbm.at[c].at[pl.ds(row0, RPT)])


_gin_agg = functools.partial(
    pl.kernel,
    out_type=jax.ShapeDtypeStruct((NC, NPAD, D), jnp.float32),
    mesh=plsc.VectorSubcoreMesh(core_axis_name="c", subcore_axis_name="s",
                                num_cores=NC, num_subcores=NS),
    scratch_types=[
        pltpu.VMEM((2, K), jnp.int32),
        pltpu.VMEM((2, K), jnp.int32),
        pltpu.VMEM((2, K), jnp.int32),
        pltpu.VMEM((2, K), jnp.int32),
        pltpu.VMEM((K, D), jnp.float32),
        pltpu.VMEM((K, D), jnp.float32),
        pltpu.VMEM_SHARED((NPAD, D), jnp.float32),
        pltpu.SemaphoreType.DMA,
        pltpu.SemaphoreType.DMA,
        pltpu.SemaphoreType.DMA,
        pltpu.SemaphoreType.DMA,
        pltpu.SemaphoreType.DMA,
        pltpu.SemaphoreType.DMA,
        pltpu.SemaphoreType.DMA,
        pltpu.SemaphoreType.DMA,
    ],
)(_agg_body)


R = 1000


def _elu(x):
    return jnp.where(x > 0, x, jnp.exp(jnp.minimum(x, 0.0)) - 1.0)


def _mlp_mid_body(p_ref, W1_ref, b1_ref, W2_ref, b2_ref, out_ref):
    z = p_ref[0] + p_ref[1]
    z = jnp.maximum(
        jnp.dot(z, W1_ref[...], preferred_element_type=jnp.float32)
        + b1_ref[...], 0.0)
    h = jnp.dot(z, W2_ref[...], preferred_element_type=jnp.float32) + b2_ref[...]
    out_ref[...] = _elu(h)


def _mlp_final_body(p_ref, W1_ref, b1_ref, W2_ref, b2_ref,
                    lw_ref, lb_ref, out_ref):
    z = p_ref[0] + p_ref[1]
    z = jnp.maximum(
        jnp.dot(z, W1_ref[...], preferred_element_type=jnp.float32)
        + b1_ref[...], 0.0)
    h = jnp.dot(z, W2_ref[...], preferred_element_type=jnp.float32) + b2_ref[...]
    h = _elu(h)
    o = jnp.dot(h, lw_ref[...], preferred_element_type=jnp.float32) + lb_ref[...]
    out_ref[...] = 1.0 / (1.0 + jnp.exp(-o))


_P_SPEC = pl.BlockSpec((NC, R, D), lambda i: (0, i, 0))
_W_SPEC = pl.BlockSpec((D, D), lambda i: (0, 0))
_B_SPEC = pl.BlockSpec((1, D), lambda i: (0, 0))

_mlp_mid = pl.pallas_call(
    _mlp_mid_body,
    grid=(N_NODES // R,),
    in_specs=[_P_SPEC, _W_SPEC, _B_SPEC, _W_SPEC, _B_SPEC],
    out_specs=pl.BlockSpec((R, D), lambda i: (i, 0)),
    out_shape=jax.ShapeDtypeStruct((N_NODES, D), jnp.float32),
)

_mlp_final = pl.pallas_call(
    _mlp_final_body,
    grid=(N_NODES // R,),
    in_specs=[_P_SPEC, _W_SPEC, _B_SPEC, _W_SPEC, _B_SPEC,
              pl.BlockSpec((D, 1), lambda i: (0, 0)),
              pl.BlockSpec((1, 1), lambda i: (0, 0))],
    out_specs=pl.BlockSpec((R, 1), lambda i: (i, 0)),
    out_shape=jax.ShapeDtypeStruct((N_NODES, 1), jnp.float32),
)


def kernel(x, edge_index, conv0_W1, conv0_b1, conv0_W2, conv0_b2,
           conv1_W1, conv1_b1, conv1_W2, conv1_b2, lin_W, lin_b):
    pad = E_PAD - N_EDGES
    pad_idx = jax.lax.iota(jnp.int32, pad)
    src_e = jnp.concatenate(
        [edge_index[0].astype(jnp.int32), pad_idx % N_NODES]
    ).reshape(NW, CPT, K)
    dst_e = jnp.concatenate(
        [edge_index[1].astype(jnp.int32),
         N_NODES + pad_idx % (NPAD - N_NODES)]
    ).reshape(NW, CPT, K)
    edges = jnp.stack([src_e, dst_e], axis=2)
    zeros = jnp.zeros((NPAD, D), jnp.float32)

    p = _gin_agg(x, edges, zeros)
    h1 = _mlp_mid(p, conv0_W1, conv0_b1.reshape(1, D),
                  conv0_W2, conv0_b2.reshape(1, D))
    p = _gin_agg(h1, edges, zeros)
    out = _mlp_final(p, conv1_W1, conv1_b1.reshape(1, D),
                     conv1_W2, conv1_b2.reshape(1, D),
                     lin_W, lin_b.reshape(1, 1))
    return out.reshape(N_NODES)


# speedup vs baseline: 2.9524x; 1.0156x over previous
import functools

import jax
import jax.numpy as jnp
from jax import lax
from jax.experimental import pallas as pl
from jax.experimental.pallas import tpu as pltpu
from jax.experimental.pallas import tpu_sc as plsc

N_NODES = 10000
NPAD = 10240
N_EDGES = 320000
D = 128

NC = 2
NS = 16
NW = NC * NS
EPW = N_EDGES // NW
K = 128
CPT = 80
E_PAD = NW * CPT * K
RPT = NPAD // NS
LAST_RPT = N_NODES - (NS - 1) * RPT


def _agg_body(h_hbm, e_hbm, zero_hbm, out_hbm,
              ei0, ei1, ei2, ei3, rows0, rows1, acc_sh,
              gs0, gs1, es0, es1, es2, es3, ss0, ss1):
    c = lax.axis_index("c")
    s = lax.axis_index("s")
    wid = s * NC + c
    row0 = s * RPT

    @pl.when(c == 0)
    def _():
        @pl.when(s < NS - 1)
        def _():
            pltpu.sync_copy(h_hbm.at[pl.ds(row0, RPT)],
                            acc_sh.at[pl.ds(row0, RPT)])

        @pl.when(s == NS - 1)
        def _():
            pltpu.sync_copy(h_hbm.at[pl.ds(row0, LAST_RPT)],
                            acc_sh.at[pl.ds(row0, LAST_RPT)])

    @pl.when(c != 0)
    def _():
        pltpu.sync_copy(zero_hbm.at[pl.ds(row0, RPT)],
                        acc_sh.at[pl.ds(row0, RPT)])

    eidx = (ei0, ei1, ei2, ei3)
    esem = (es0, es1, es2, es3)
    rows = (rows0, rows1)
    gsem = (gs0, gs1)
    ssem = (ss0, ss1)
    my_e = e_hbm.at[wid]

    pltpu.sync_copy(my_e.at[0], eidx[0])
    pltpu.async_copy(h_hbm.at[eidx[0].at[0]], rows[0], gsem[0])
    pltpu.async_copy(my_e.at[1], eidx[1], esem[1])
    plsc.subcore_barrier()

    def group(g, carry):
        for b4 in range(4):
            i = g * 4 + b4
            b = b4 % 2
            o = 1 - b

            pltpu.make_async_copy(
                h_hbm.at[eidx[b4].at[0]], rows[b], gsem[b]).wait()

            @pl.when(i >= 1)
            def _():
                pltpu.make_async_copy(
                    rows[o], acc_sh.at[eidx[(b4 - 1) % 4].at[1]],
                    ssem[o]).wait()

            @pl.when(i + 1 < CPT)
            def _():
                pltpu.make_async_copy(my_e.at[i + 1], eidx[(b4 + 1) % 4],
                                      esem[(b4 + 1) % 4]).wait()
                pltpu.async_copy(h_hbm.at[eidx[(b4 + 1) % 4].at[0]],
                                 rows[o], gsem[o])

            pltpu.async_copy(rows[b], acc_sh.at[eidx[b4].at[1]],
                             ssem[b], add=True)

            @pl.when(i + 2 < CPT)
            def _():
                pltpu.async_copy(my_e.at[i + 2], eidx[(b4 + 2) % 4],
                                 esem[(b4 + 2) % 4])
        return carry

    lax.fori_loop(0, CPT // 4, group, 0)
    pltpu.make_async_copy(
        rows[(CPT - 1) % 2], acc_sh.at[eidx[(CPT - 1) % 4].at[1]],
        ssem[(CPT - 1) % 2]).wait()
    plsc.subcore_barrier()
    pltpu.sync_copy(acc_sh.at[pl.ds(row0, RPT)],
                    out_hbm.at[c].at[pl.ds(row0, RPT)])


_gin_agg = functools.partial(
    pl.kernel,
    out_type=jax.ShapeDtypeStruct((NC, NPAD, D), jnp.float32),
    mesh=plsc.VectorSubcoreMesh(core_axis_name="c", subcore_axis_name="s",
                                num_cores=NC, num_subcores=NS),
    scratch_types=[
        pltpu.VMEM((2, K), jnp.int32),
        pltpu.VMEM((2, K), jnp.int32),
        pltpu.VMEM((2, K), jnp.int32),
        pltpu.VMEM((2, K), jnp.int32),
        pltpu.VMEM((K, D), jnp.float32),
        pltpu.VMEM((K, D), jnp.float32),
        pltpu.VMEM_SHARED((NPAD, D), jnp.float32),
        pltpu.SemaphoreType.DMA,
        pltpu.SemaphoreType.DMA,
        pltpu.SemaphoreType.DMA,
        pltpu.SemaphoreType.DMA,
        pltpu.SemaphoreType.DMA,
        pltpu.SemaphoreType.DMA,
        pltpu.SemaphoreType.DMA,
        pltpu.SemaphoreType.DMA,
    ],
)(_agg_body)


R = 2000


def _elu(x):
    return jnp.where(x > 0, x, jnp.exp(jnp.minimum(x, 0.0)) - 1.0)


def _mlp_mid_body(p_ref, W1_ref, b1_ref, W2_ref, b2_ref, out_ref):
    z = p_ref[0] + p_ref[1]
    z = jnp.maximum(
        jnp.dot(z, W1_ref[...], preferred_element_type=jnp.float32)
        + b1_ref[...], 0.0)
    h = jnp.dot(z, W2_ref[...], preferred_element_type=jnp.float32) + b2_ref[...]
    out_ref[...] = _elu(h)


def _mlp_final_body(p_ref, W1_ref, b1_ref, W2_ref, b2_ref,
                    lw_ref, lb_ref, out_ref):
    z = p_ref[0] + p_ref[1]
    z = jnp.maximum(
        jnp.dot(z, W1_ref[...], preferred_element_type=jnp.float32)
        + b1_ref[...], 0.0)
    h = jnp.dot(z, W2_ref[...], preferred_element_type=jnp.float32) + b2_ref[...]
    h = _elu(h)
    o = jnp.dot(h, lw_ref[...], preferred_element_type=jnp.float32) + lb_ref[...]
    out_ref[...] = 1.0 / (1.0 + jnp.exp(-o))


_P_SPEC = pl.BlockSpec((NC, R, D), lambda i: (0, i, 0))
_W_SPEC = pl.BlockSpec((D, D), lambda i: (0, 0))
_B_SPEC = pl.BlockSpec((1, D), lambda i: (0, 0))

_mlp_mid = pl.pallas_call(
    _mlp_mid_body,
    grid=(N_NODES // R,),
    in_specs=[_P_SPEC, _W_SPEC, _B_SPEC, _W_SPEC, _B_SPEC],
    out_specs=pl.BlockSpec((R, D), lambda i: (i, 0)),
    out_shape=jax.ShapeDtypeStruct((N_NODES, D), jnp.float32),
)

_mlp_final = pl.pallas_call(
    _mlp_final_body,
    grid=(N_NODES // R,),
    in_specs=[_P_SPEC, _W_SPEC, _B_SPEC, _W_SPEC, _B_SPEC,
              pl.BlockSpec((D, 1), lambda i: (0, 0)),
              pl.BlockSpec((1, 1), lambda i: (0, 0))],
    out_specs=pl.BlockSpec((R, 1), lambda i: (i, 0)),
    out_shape=jax.ShapeDtypeStruct((N_NODES, 1), jnp.float32),
)


def kernel(x, edge_index, conv0_W1, conv0_b1, conv0_W2, conv0_b2,
           conv1_W1, conv1_b1, conv1_W2, conv1_b2, lin_W, lin_b):
    pad = E_PAD - N_EDGES
    pad_idx = jax.lax.iota(jnp.int32, pad)
    src_e = jnp.concatenate(
        [edge_index[0].astype(jnp.int32), pad_idx % N_NODES]
    ).reshape(NW, CPT, K)
    dst_e = jnp.concatenate(
        [edge_index[1].astype(jnp.int32),
         N_NODES + pad_idx % (NPAD - N_NODES)]
    ).reshape(NW, CPT, K)
    edges = jnp.stack([src_e, dst_e], axis=2)
    zeros = jnp.zeros((NPAD, D), jnp.float32)

    p = _gin_agg(x, edges, zeros)
    h1 = _mlp_mid(p, conv0_W1, conv0_b1.reshape(1, D),
                  conv0_W2, conv0_b2.reshape(1, D))
    p = _gin_agg(h1, edges, zeros)
    out = _mlp_final(p, conv1_W1, conv1_b1.reshape(1, D),
                     conv1_W2, conv1_b2.reshape(1, D),
                     lin_W, lin_b.reshape(1, 1))
    return out.reshape(N_NODES)
